# baseline (speedup 1.0000x reference)
import functools

import jax
import jax.numpy as jnp
from jax import lax
from jax.experimental import pallas as pl
from jax.experimental.pallas import tpu as pltpu
from jax.experimental.pallas import tpu_sc as plsc

D = 64
LANES = 16
NC, NS = 2, 16
NW = NC * NS
CHUNK = 512
SUB = 128
SUBS = CHUNK // SUB
BB = 512


def _make_sc_gather(b, h):
    n_rows = b * h
    rows_per_w = n_rows // NW
    n_chunks = rows_per_w // CHUNK
    assert n_chunks == h and b == NW * CHUNK and n_chunks % 2 == 0
    idx_rows_w = rows_per_w // SUB
    mesh = plsc.VectorSubcoreMesh(
        core_axis_name="c", subcore_axis_name="s", num_cores=NC, num_subcores=NS
    )

    @functools.partial(
        pl.kernel,
        out_type=jax.ShapeDtypeStruct((h, b, D), jnp.float32),
        mesh=mesh,
        compiler_params=pltpu.CompilerParams(
            needs_layout_passes=False, use_tc_tiling_on_sc=False
        ),
        scratch_types=[
            pltpu.VMEM((idx_rows_w, SUB), jnp.int32),
            pltpu.VMEM((CHUNK, D), jnp.float32),
            pltpu.VMEM((CHUNK, D), jnp.float32),
            pltpu.SemaphoreType.DMA,
            pltpu.SemaphoreType.DMA,
            pltpu.SemaphoreType.DMA,
            pltpu.SemaphoreType.DMA,
        ],
    )
    def gather_rows(
        table_hbm, idx_hbm, out_hbm,
        idx_v, rows0, rows1, gsem0, gsem1, wsem0, wsem1,
    ):
        wid = lax.axis_index("s") * NC + lax.axis_index("c")
        b0 = wid * CHUNK

        def pf_body(i, _):
            pltpu.async_copy(
                idx_hbm.at[pl.ds(i * (b // SUB) + wid * SUBS, SUBS)],
                idx_v.at[pl.ds(i * SUBS, SUBS)],
                gsem0,
            )
            return 0

        lax.fori_loop(0, n_chunks, pf_body, 0)
        pltpu.make_async_copy(
            idx_hbm.at[pl.ds(0, idx_rows_w)], idx_v, gsem0
        ).wait()

        bufs = (rows0, rows1)
        gsems = (gsem0, gsem1)
        wsems = (wsem0, wsem1)

        def issue_gather(ci, buf, gsem):
            for j in range(SUBS):
                pltpu.async_copy(
                    table_hbm.at[idx_v.at[ci * SUBS + j]],
                    buf.at[pl.ds(j * SUB, SUB)],
                    gsem,
                )

        def drain_gather(ci, buf, gsem):
            for j in range(SUBS):
                pltpu.make_async_copy(
                    table_hbm.at[idx_v.at[ci * SUBS + j]],
                    buf.at[pl.ds(j * SUB, SUB)],
                    gsem,
                ).wait()

        out_dummy = out_hbm.at[0, pl.ds(0, CHUNK)]

        issue_gather(0, rows0, gsem0)

        def step(k, _):
            for p in range(2):
                i = 2 * k + p
                buf, gsem, wsem = bufs[p], gsems[p], wsems[p]
                nbuf, ngsem, nwsem = bufs[1 - p], gsems[1 - p], wsems[1 - p]

                @pl.when(i + 1 < n_chunks)
                def _():
                    @pl.when(i >= 1)
                    def _():
                        pltpu.make_async_copy(nbuf, out_dummy, nwsem).wait()

                    issue_gather(i + 1, nbuf, ngsem)

                drain_gather(i, buf, gsem)
                pltpu.async_copy(
                    buf, out_hbm.at[i, pl.ds(b0, CHUNK)], wsem
                )
            return 0

        lax.fori_loop(0, n_chunks // 2, step, 0)
        pltpu.make_async_copy(rows0, out_dummy, wsem0).wait()
        pltpu.make_async_copy(rows1, out_dummy, wsem1).wait()

    return gather_rows


def _tc_norm_body(g_ref, o_ref):
    x = g_ref[0]
    n = jnp.sqrt(jnp.sum(x * x, axis=1, keepdims=True))
    scaled = x / jnp.maximum(n, 1e-12)
    o_ref[0] = scaled.T


def _make_tc_normalize(b, h):
    return pl.pallas_call(
        _tc_norm_body,
        grid=(b // BB, h),
        in_specs=[
            pl.BlockSpec((1, BB, D), lambda i, j: (j, i, 0)),
        ],
        out_specs=pl.BlockSpec((1, D, BB), lambda i, j: (j, 0, i)),
        out_shape=jax.ShapeDtypeStruct((h, D, b), jnp.float32),
    )


def kernel(inputs, weights):
    b, h = inputs.shape
    idx = inputs.T.reshape(b * h // SUB, SUB).astype(jnp.int32)
    g = _make_sc_gather(b, h)(weights, idx)
    out_t = _make_tc_normalize(b, h)(g)
    return jnp.transpose(out_t, (2, 0, 1))


# speedup vs baseline: 1.3822x; 1.3822x over previous
import functools

import jax
import jax.numpy as jnp
from jax import lax
from jax.experimental import pallas as pl
from jax.experimental.pallas import tpu as pltpu
from jax.experimental.pallas import tpu_sc as plsc

D = 64
LANES = 16
NC, NS = 2, 16
NW = NC * NS
CHUNK = 512
SUB = 128
SUBS = CHUNK // SUB


def _normalize_chunk(rows_v, sbuf, n_groups):
    lane = lax.iota(jnp.int32, LANES)

    def group_body(g, _):
        r0 = g * LANES
        for rl in range(LANES):
            r = r0 + rl
            v0 = rows_v[r, pl.ds(0, LANES)]
            v1 = rows_v[r, pl.ds(LANES, LANES)]
            v2 = rows_v[r, pl.ds(2 * LANES, LANES)]
            v3 = rows_v[r, pl.ds(3 * LANES, LANES)]
            sbuf[rl, :] = (v0 * v0 + v1 * v1) + (v2 * v2 + v3 * v3)
        acc = jnp.zeros((LANES,), jnp.float32)
        for c in range(LANES):
            cv = lax.bitwise_and(lane + c, LANES - 1)
            acc = acc + plsc.load_gather(sbuf, [lane, cv])
        i = lax.bitcast_convert_type(acc, jnp.int32)
        i = 0x5F3759DF - lax.shift_right_logical(i, 1)
        y = lax.bitcast_convert_type(i, jnp.float32)
        xh = acc * 0.5
        for _ in range(3):
            y = y * (1.5 - xh * y * y)
        y = jnp.minimum(y, 1e12)
        for rl in range(LANES):
            r = r0 + rl
            yb = lax.broadcast(y[rl], (LANES,))
            for q in range(4):
                sl = pl.ds(q * LANES, LANES)
                rows_v[r, sl] = rows_v[r, sl] * yb
        return 0

    lax.fori_loop(0, n_groups, group_body, 0)


def _make_sc_kernel(n_rows):
    rows_per_w = n_rows // NW
    n_chunks = rows_per_w // CHUNK
    idx_rows_w = rows_per_w // SUB
    assert n_chunks % 2 == 0
    mesh = plsc.VectorSubcoreMesh(
        core_axis_name="c", subcore_axis_name="s", num_cores=NC, num_subcores=NS
    )

    @functools.partial(
        pl.kernel,
        out_type=jax.ShapeDtypeStruct((n_rows, D), jnp.float32),
        mesh=mesh,
        compiler_params=pltpu.CompilerParams(
            needs_layout_passes=False, use_tc_tiling_on_sc=False
        ),
        scratch_types=[
            pltpu.VMEM((idx_rows_w, SUB), jnp.int32),
            pltpu.VMEM((CHUNK, D), jnp.float32),
            pltpu.VMEM((CHUNK, D), jnp.float32),
            pltpu.VMEM((LANES, LANES), jnp.float32),
            pltpu.SemaphoreType.DMA,
            pltpu.SemaphoreType.DMA,
            pltpu.SemaphoreType.DMA,
            pltpu.SemaphoreType.DMA,
        ],
    )
    def gather_norm(
        table_hbm, idx_hbm, out_hbm,
        idx_v, rows0, rows1, sbuf, gsem0, gsem1, wsem0, wsem1,
    ):
        wid = lax.axis_index("s") * NC + lax.axis_index("c")
        row0 = wid * rows_per_w

        pltpu.sync_copy(idx_hbm.at[pl.ds(wid * idx_rows_w, idx_rows_w)], idx_v)

        bufs = (rows0, rows1)
        gsems = (gsem0, gsem1)
        wsems = (wsem0, wsem1)

        def issue_gather(ci, buf, gsem):
            for j in range(SUBS):
                pltpu.async_copy(
                    table_hbm.at[idx_v.at[ci * SUBS + j]],
                    buf.at[pl.ds(j * SUB, SUB)],
                    gsem,
                )

        def drain_gather(ci, buf, gsem):
            for j in range(SUBS):
                pltpu.make_async_copy(
                    table_hbm.at[idx_v.at[ci * SUBS + j]],
                    buf.at[pl.ds(j * SUB, SUB)],
                    gsem,
                ).wait()

        out_dummy = out_hbm.at[pl.ds(0, CHUNK)]

        issue_gather(0, rows0, gsem0)

        def step(k, _):
            for b in range(2):
                i = 2 * k + b
                buf, gsem, wsem = bufs[b], gsems[b], wsems[b]
                nbuf, ngsem, nwsem = bufs[1 - b], gsems[1 - b], wsems[1 - b]

                @pl.when(i + 1 < n_chunks)
                def _():
                    @pl.when(i >= 1)
                    def _():
                        pltpu.make_async_copy(nbuf, out_dummy, nwsem).wait()

                    issue_gather(i + 1, nbuf, ngsem)

                drain_gather(i, buf, gsem)
                _normalize_chunk(buf, sbuf, CHUNK // LANES)
                pltpu.async_copy(
                    buf, out_hbm.at[pl.ds(row0 + i * CHUNK, CHUNK)], wsem
                )
            return 0

        lax.fori_loop(0, n_chunks // 2, step, 0)
        pltpu.make_async_copy(rows0, out_dummy, wsem0).wait()
        pltpu.make_async_copy(rows1, out_dummy, wsem1).wait()

    return gather_norm


def kernel(inputs, weights):
    b, h = inputs.shape
    n_rows = b * h
    idx = inputs.reshape(n_rows // SUB, SUB).astype(jnp.int32)
    out = _make_sc_kernel(n_rows)(weights, idx)
    return out.reshape(b, h, D)
